# baseline (speedup 1.0000x reference)
import functools

import jax
import jax.numpy as jnp
import numpy as np
from jax import lax
from jax.experimental import pallas as pl
from jax.experimental.pallas import tpu as pltpu
from jax.experimental.pallas import tpu_sc as plsc

B = 4
N = 4096
NSLOT = 2 * B
NELEM = 3 * N
QB = 256
N_BISECT = 48

_QS = (0.05, 0.95, 0.25, 0.75)
_IDXF = [np.float32(q) * np.float32(NELEM - 1) for q in _QS]
_KLO = [int(np.floor(i)) for i in _IDXF]
_FRAC = [np.float32(i - np.floor(i)) for i in _IDXF]
_RANKS = []
for _k in _KLO:
    _RANKS.extend([_k, _k + 1])


def _stage1_body(qref, ktref, rref):
    q = qref[0]
    kt = ktref[0]
    qx, qy, qz = q[:, 0:1], q[:, 1:2], q[:, 2:3]
    kx, ky, kz = kt[0:1, :], kt[1:2, :], kt[2:3, :]
    n1 = jnp.sum(q * q, axis=1, keepdims=True)
    n2 = jnp.sum(kt * kt, axis=0, keepdims=True)
    dot = jax.lax.dot_general(
        q.astype(jnp.bfloat16), kt.astype(jnp.bfloat16),
        (((1,), (0,)), ((), ())),
        preferred_element_type=jnp.float32)
    d2 = (n1 + n2) - 2.0 * dot
    m = jnp.min(d2, axis=1, keepdims=True)
    ii = jax.lax.broadcasted_iota(jnp.int32, (QB, N), 1)
    idx = jnp.min(jnp.where(d2 == m, ii, N), axis=1, keepdims=True)
    sel = ii == idx
    cx = jnp.sum(jnp.where(sel, kx, 0.0), axis=1, keepdims=True)
    cy = jnp.sum(jnp.where(sel, ky, 0.0), axis=1, keepdims=True)
    cz = jnp.sum(jnp.where(sel, kz, 0.0), axis=1, keepdims=True)
    rref[0] = jnp.concatenate([qx - cx, qy - cy, qz - cz], axis=1)


CHUNKS = NELEM // 16
NBIS_SC = 36
_RKF = tuple(float(k) for k in _KLO)
_FRF = tuple(float(f) for f in _FRAC)


def _sel4(j, vals):
    return jnp.where(j == 0, vals[0],
                     jnp.where(j == 1, vals[1],
                               jnp.where(j == 2, vals[2], vals[3])))


_mesh = plsc.VectorSubcoreMesh(core_axis_name="c", subcore_axis_name="s")


@functools.partial(
    pl.kernel, mesh=_mesh,
    out_type=jax.ShapeDtypeStruct((512,), jnp.float32),
    scratch_types=[pltpu.VMEM((NELEM,), jnp.float32),
                   pltpu.VMEM((16,), jnp.float32)],
)
def _sc_select(r_hbm, out_hbm, vbuf, obuf):
    cid = lax.axis_index("c")
    sid = lax.axis_index("s")
    w = sid * 2 + cid
    slot = w // 4
    j = w % 4
    pltpu.sync_copy(r_hbm.at[pl.ds(slot * NELEM, NELEM)], vbuf)

    rank = _sel4(j, (614, 11672, 3071, 9215))
    frac = _sel4(j, _FRF)

    def bis(_, carry):
        lo0, hi0, lo1, hi1 = carry
        mid0 = 0.5 * (lo0 + hi0)
        mid1 = 0.5 * (lo1 + hi1)

        def cnt_body(t, acc):
            a0, a1 = acc
            c = vbuf[pl.ds(t * 16, 16)]
            a0 = a0 + jnp.where(c <= mid0, 1, 0)
            a1 = a1 + jnp.where(c <= mid1, 1, 0)
            return a0, a1

        z = jnp.full((16,), 0, jnp.int32)
        a0, a1 = lax.fori_loop(0, CHUNKS, cnt_body, (z, z))
        tot0 = a0[0]
        tot1 = a1[0]
        for l in range(1, 16):
            tot0 = tot0 + a0[l]
            tot1 = tot1 + a1[l]
        p0 = tot0 >= rank + 1
        p1 = tot1 >= rank + 2
        return (jnp.where(p0, lo0, mid0), jnp.where(p0, mid0, hi0),
                jnp.where(p1, lo1, mid1), jnp.where(p1, mid1, hi1))

    lo0, hi0, lo1, hi1 = lax.fori_loop(
        0, NBIS_SC, bis, (-1.0, 1.0, -1.0, 1.0))
    qv = hi0 * (1.0 - frac) + hi1 * frac
    obuf[...] = jnp.broadcast_to(qv, (16,))
    pltpu.sync_copy(obuf, out_hbm.at[pl.ds(w * 16, 16)])


def _stage3_body(rref, qref, bref, eref):
    beg_stds = []
    end_stds = []
    for s in range(NSLOT):
        v = rref[s]
        q05 = qref[s, 0]
        q95 = qref[s, 1]
        q25 = qref[s, 2]
        q75 = qref[s, 3]
        for thr_mask, acc in (((v < q05) | (v > q95), beg_stds),
                              ((v > q25) & (v < q75), end_stds)):
            m = thr_mask.astype(jnp.float32)
            n = jnp.sum(m)
            mean = jnp.sum(v * m) / n
            var = jnp.sum(((v - mean) ** 2) * m) / (n - 1.0)
            acc.append(jnp.sqrt(var))
    beg = 0.0
    end = 0.0
    for b in range(B):
        beg += jnp.maximum(beg_stds[2 * b], beg_stds[2 * b + 1])
        end += jnp.maximum(end_stds[2 * b], end_stds[2 * b + 1])
    bref[...] = jnp.broadcast_to(beg / B, (1, 1))
    eref[...] = jnp.broadcast_to(end / B, (1, 1))


@jax.jit
def kernel(x, y):
    q_all = jnp.stack([x, y], axis=1).reshape(NSLOT, N, 3)
    k_all = jnp.stack([y, x], axis=1).reshape(NSLOT, N, 3)
    kt_all = k_all.transpose(0, 2, 1)

    resid = pl.pallas_call(
        _stage1_body,
        grid=(NSLOT, N // QB),
        in_specs=[
            pl.BlockSpec((1, QB, 3), lambda s, qb: (s, qb, 0)),
            pl.BlockSpec((1, 3, N), lambda s, qb: (s, 0, 0)),
        ],
        out_specs=pl.BlockSpec((1, QB, 3), lambda s, qb: (s, qb, 0)),
        out_shape=jax.ShapeDtypeStruct((NSLOT, N, 3), jnp.float32),
    )(q_all, kt_all)

    sc_out = _sc_select(resid.reshape(-1))
    qmat = sc_out.reshape(32, 16)[:, 0].reshape(NSLOT, 4)

    r_flat = resid.reshape(NSLOT, NELEM // 128, 128)
    beg, end = pl.pallas_call(
        _stage3_body,
        in_specs=[
            pl.BlockSpec(memory_space=pltpu.VMEM),
            pl.BlockSpec(memory_space=pltpu.SMEM),
        ],
        out_shape=(jax.ShapeDtypeStruct((1, 1), jnp.float32),
                   jax.ShapeDtypeStruct((1, 1), jnp.float32)),
    )(r_flat, qmat)
    return (beg[0, 0], end[0, 0])
